# baseline (speedup 1.0000x reference)
import functools

import jax
import jax.numpy as jnp
from jax import lax
from jax.experimental import pallas as pl
from jax.experimental.pallas import tpu as pltpu
from jax.experimental.pallas import tpu_sc as plsc

NUM_USERS = 100000
EMBED_DIM = 64
BATCH = 4096

NC = 2
NS = 16
L = 16
NW = NC * NS
B_PER_W = BATCH // NW
D_CHUNKS = EMBED_DIM // L


def _tae_kernel(users_hbm, ts_hbm, table_hbm, w_hbm, b_hbm, out_hbm,
                idx_v, t_v, rows_v, w_v, bias_v, sem):
    wid = lax.axis_index("s") * NC + lax.axis_index("c")
    base = wid * B_PER_W

    pltpu.sync_copy(users_hbm.at[pl.ds(base, B_PER_W)], idx_v)
    pltpu.sync_copy(ts_hbm.at[pl.ds(base, B_PER_W)], t_v)
    pltpu.sync_copy(w_hbm, w_v)
    pltpu.sync_copy(b_hbm, bias_v)

    pltpu.async_copy(table_hbm.at[idx_v], rows_v, sem).wait()

    w_chunks = [w_v[pl.ds(c * L, L)] for c in range(D_CHUNKS)]
    bias_chunks = [bias_v[pl.ds(c * L, L)] for c in range(D_CHUNKS)]

    def body(g, carry):
        t_chunk = t_v[pl.ds(g * L, L)]
        for j in range(L):
            b = g * L + j
            tb = t_chunk.at[jnp.full((L,), j, jnp.int32)].get(
                mode="promise_in_bounds")
            for c in range(D_CHUNKS):
                sl = pl.ds(c * L, L)
                rows_v[b, sl] = (rows_v[b, sl] + tb * w_chunks[c]
                                 + bias_chunks[c])
        return carry
    lax.fori_loop(0, B_PER_W // L, body, 0)

    pltpu.sync_copy(rows_v, out_hbm.at[pl.ds(base, B_PER_W)])


@jax.jit
def _run(users, timestamps, table, w_flat, time_b):
    mesh = plsc.VectorSubcoreMesh(core_axis_name="c", subcore_axis_name="s",
                                  num_cores=NC)
    return pl.kernel(
        _tae_kernel,
        out_type=jax.ShapeDtypeStruct((BATCH, EMBED_DIM), jnp.float32),
        mesh=mesh,
        compiler_params=pltpu.CompilerParams(use_tc_tiling_on_sc=False),
        scratch_types=[
            pltpu.VMEM((B_PER_W,), jnp.int32),
            pltpu.VMEM((B_PER_W,), jnp.float32),
            pltpu.VMEM((B_PER_W, EMBED_DIM), jnp.float32),
            pltpu.VMEM((EMBED_DIM,), jnp.float32),
            pltpu.VMEM((EMBED_DIM,), jnp.float32),
            pltpu.SemaphoreType.DMA,
        ],
    )(users, timestamps, table, w_flat, time_b)


def kernel(users, timestamps, table, time_w, time_b):
    return _run(users.astype(jnp.int32), timestamps, table,
                time_w.reshape(EMBED_DIM), time_b)
